# baseline (speedup 1.0000x reference)
import jax
from jax.experimental import pallas as pl
from jax.experimental.pallas import tpu as pltpu

_N_STREAMS = 4


def _dma_copy_rows(sl, d, dtype, nstreams):
    rows = sl // nstreams

    def body(in_ref, out_ref, buf, in_sems, out_sems):
        fetches = [
            pltpu.make_async_copy(
                in_ref.at[pl.ds(i * rows, rows)], buf.at[i], in_sems.at[i]
            )
            for i in range(nstreams)
        ]
        flushes = [
            pltpu.make_async_copy(
                buf.at[i], out_ref.at[pl.ds(i * rows, rows)], out_sems.at[i]
            )
            for i in range(nstreams)
        ]
        for c in fetches:
            c.start()
        for i in range(nstreams):
            fetches[i].wait()
            flushes[i].start()
        for c in flushes:
            c.wait()

    return pl.pallas_call(
        body,
        in_specs=[pl.BlockSpec(memory_space=pl.ANY)],
        out_specs=pl.BlockSpec(memory_space=pl.ANY),
        scratch_shapes=[
            pltpu.VMEM((nstreams, rows, d), dtype),
            pltpu.SemaphoreType.DMA((nstreams,)),
            pltpu.SemaphoreType.DMA((nstreams,)),
        ],
        out_shape=jax.ShapeDtypeStruct((sl, d), dtype),
    )


def kernel(x, emb_weight):
    sl = x.shape[1]
    _, d = emb_weight.shape
    nstreams = _N_STREAMS
    while sl % nstreams:
        nstreams //= 2
    return _dma_copy_rows(sl, d, emb_weight.dtype, nstreams)(emb_weight)
